# Initial kernel scaffold; baseline (speedup 1.0000x reference)
import jax
import jax.numpy as jnp
from jax.experimental import pallas as pl


def kernel(x, edge_index, edge_attr, W_msg, W_edge, W_upd, b_upd):
    raise NotImplementedError("write your pallas kernel here")


# speedup vs baseline: 1.7509x; 1.7509x over previous
import functools

import jax
import jax.numpy as jnp
from jax import lax
from jax.experimental import pallas as pl
from jax.experimental.pallas import tpu as pltpu
from jax.experimental.pallas import tpu_sc as plsc

N = 10000
E = 320000
D = 128
HD = 64
NC = 2
NS = 16
EPS = E // NS
C = 80
KPS = EPS // C


def _split_mm_body(a_ref, w_ref, o_ref):
    a = a_ref[...]
    o_ref[0] = jnp.dot(a, w_ref[:, :HD], preferred_element_type=jnp.float32)
    o_ref[1] = jnp.dot(a, w_ref[:, HD:], preferred_element_type=jnp.float32)


def _split_mm(a, w, blk):
    r = a.shape[0]
    return pl.pallas_call(
        _split_mm_body,
        grid=(r // blk,),
        in_specs=[pl.BlockSpec((blk, D), lambda i: (i, 0)),
                  pl.BlockSpec((D, D), lambda i: (0, 0))],
        out_specs=pl.BlockSpec((2, blk, HD), lambda i: (0, i, 0)),
        out_shape=jax.ShapeDtypeStruct((2, r, HD), jnp.float32),
    )(a, w)


def _h2_mm_body(ea_ref, m_ref, w_ref, o_ref):
    pv = ea_ref[...] + jnp.concatenate([m_ref[0], m_ref[1]], axis=-1)
    o_ref[0] = jnp.dot(pv, w_ref[:, :HD], preferred_element_type=jnp.float32)
    o_ref[1] = jnp.dot(pv, w_ref[:, HD:], preferred_element_type=jnp.float32)


def _h2_mm(ea, m, w, blk):
    return pl.pallas_call(
        _h2_mm_body,
        grid=(E // blk,),
        in_specs=[pl.BlockSpec((blk, D), lambda i: (i, 0)),
                  pl.BlockSpec((2, blk, HD), lambda i: (0, i, 0)),
                  pl.BlockSpec((D, D), lambda i: (0, 0))],
        out_specs=pl.BlockSpec((2, blk, HD), lambda i: (0, i, 0)),
        out_shape=jax.ShapeDtypeStruct((2, E, HD), jnp.float32),
    )(ea, m, w)


def _upd_body(x_ref, agg_ref, wu_ref, b_ref, wm_ref, xo_ref, yo_ref):
    up = (jnp.dot(agg_ref[0], wu_ref[:HD, :], preferred_element_type=jnp.float32)
          + jnp.dot(agg_ref[1], wu_ref[HD:, :], preferred_element_type=jnp.float32)
          + b_ref[...])
    xn = x_ref[...] + jnp.maximum(up, 0.0)
    xo_ref[...] = xn
    yo_ref[0] = jnp.dot(xn, wm_ref[:, :HD], preferred_element_type=jnp.float32)
    yo_ref[1] = jnp.dot(xn, wm_ref[:, HD:], preferred_element_type=jnp.float32)


def _update(x, agg, wu, b2, wm, blk=2000):
    return pl.pallas_call(
        _upd_body,
        grid=(N // blk,),
        in_specs=[pl.BlockSpec((blk, D), lambda i: (i, 0)),
                  pl.BlockSpec((2, blk, HD), lambda i: (0, i, 0)),
                  pl.BlockSpec((D, D), lambda i: (0, 0)),
                  pl.BlockSpec((1, D), lambda i: (0, 0)),
                  pl.BlockSpec((D, D), lambda i: (0, 0))],
        out_specs=[pl.BlockSpec((blk, D), lambda i: (i, 0)),
                   pl.BlockSpec((2, blk, HD), lambda i: (0, i, 0))],
        out_shape=[jax.ShapeDtypeStruct((N, D), jnp.float32),
                   jax.ShapeDtypeStruct((2, N, HD), jnp.float32)],
    )(x, agg, wu, b2, wm)


def _sc_body(with_m, *refs):
    if with_m:
        (y_hbm, h_hbm, src_hbm, dst_hbm, z_hbm,
         agg_hbm, m_hbm,
         srci, dsti, gv, hv, aggsh, sg, sh) = refs
    else:
        (y_hbm, h_hbm, src_hbm, dst_hbm, z_hbm,
         agg_hbm,
         srci, dsti, gv, hv, aggsh, sg, sh) = refs

    c = lax.axis_index("c")
    s = lax.axis_index("s")

    @pl.when(s == 0)
    def _zero():
        pltpu.sync_copy(z_hbm, aggsh)
    plsc.subcore_barrier()

    pltpu.sync_copy(src_hbm.at[s], srci)
    pltpu.sync_copy(dst_hbm.at[s], dsti)
    off = c * N

    def fix(r, cr):
        for j in range(C // 16):
            sl = pl.ds(j * 16, 16)
            srci[r, sl] = srci[r, sl] + off
        return cr
    lax.fori_loop(0, KPS, fix, 0)

    def chunk(k, carry):
        base = s * EPS + k * C
        gcp = pltpu.async_copy(y_hbm.at[srci.at[k]], gv, sg)
        hcp = pltpu.async_copy(h_hbm.at[c, pl.ds(base, C), :], hv, sh)
        gcp.wait()
        hcp.wait()

        def row(r, cr):
            for j in range(HD // 16):
                sl = pl.ds(j * 16, 16)
                gv[r, sl] = jnp.maximum(gv[r, sl] + hv[r, sl], 0.0)
            return cr
        lax.fori_loop(0, C, row, 0)

        pltpu.sync_copy(gv, aggsh.at[dsti.at[k]], add=True)
        if with_m:
            pltpu.sync_copy(gv, m_hbm.at[c, pl.ds(base, C), :])
        return carry

    lax.fori_loop(0, KPS, chunk, 0)
    plsc.subcore_barrier()

    @pl.when(s < NS - 1)
    def _dump_main():
        pltpu.sync_copy(aggsh.at[pl.ds(s * 640, 640), :],
                        agg_hbm.at[c, pl.ds(s * 640, 640), :])

    @pl.when(s == NS - 1)
    def _dump_tail():
        pltpu.sync_copy(aggsh.at[pl.ds(9600, 400), :],
                        agg_hbm.at[c, pl.ds(9600, 400), :])


_MESH = plsc.VectorSubcoreMesh(core_axis_name="c", subcore_axis_name="s",
                               num_cores=NC, num_subcores=NS)

_SC_SCRATCH = [
    pltpu.VMEM((KPS, C), jnp.int32),
    pltpu.VMEM((KPS, C), jnp.int32),
    pltpu.VMEM((C, HD), jnp.float32),
    pltpu.VMEM((C, HD), jnp.float32),
    pltpu.VMEM_SHARED((N, HD), jnp.float32),
    pltpu.SemaphoreType.DMA,
    pltpu.SemaphoreType.DMA,
]

_SC_PARAMS = pltpu.CompilerParams(use_tc_tiling_on_sc=False)

_sc_m = pl.kernel(
    functools.partial(_sc_body, True),
    out_type=[jax.ShapeDtypeStruct((NC, N, HD), jnp.float32),
              jax.ShapeDtypeStruct((NC, E, HD), jnp.float32)],
    mesh=_MESH,
    scratch_types=_SC_SCRATCH,
    compiler_params=_SC_PARAMS,
)

_sc_nom = pl.kernel(
    functools.partial(_sc_body, False),
    out_type=jax.ShapeDtypeStruct((NC, N, HD), jnp.float32),
    mesh=_MESH,
    scratch_types=_SC_SCRATCH,
    compiler_params=_SC_PARAMS,
)


def kernel(x, edge_index, edge_attr, W_msg, W_edge, W_upd, b_upd):
    src = edge_index[0].astype(jnp.int32).reshape(NS, KPS, C)
    dst = edge_index[1].astype(jnp.int32).reshape(NS, KPS, C)
    zeros = jnp.zeros((N, HD), jnp.float32)
    b2 = b_upd.reshape(1, D)

    y1 = _split_mm(x, W_msg, 2000).reshape(NC * N, HD)
    h1 = _split_mm(edge_attr, W_edge, 2560)
    agg1, m1 = _sc_m(y1, h1, src, dst, zeros)
    x1, y2 = _update(x, agg1, W_upd, b2, W_msg)

    h2 = _h2_mm(edge_attr, m1, W_edge, 2560)
    agg2 = _sc_nom(y2.reshape(NC * N, HD), h2, src, dst, zeros)
    x2, _ = _update(x1, agg2, W_upd, b2, W_msg)
    return x2
